# baseline (speedup 1.0000x reference)
import functools
import math

import jax
import jax.numpy as jnp
from jax import lax
from jax.experimental import pallas as pl
from jax.experimental.pallas import tpu as pltpu
from jax.experimental.pallas import tpu_sc as plsc

EMB = 32
SCALE = math.sqrt(32.0)

B = 16384 * 50
NW = 32
B_PER_W = B // NW
CHUNK = 1024
N_CHUNKS = B_PER_W // CHUNK

_mesh = plsc.VectorSubcoreMesh(core_axis_name="c", subcore_axis_name="s")


@functools.partial(
    pl.kernel,
    mesh=_mesh,
    out_type=jax.ShapeDtypeStruct((B, EMB), jnp.float32),
    scratch_types=[
        pltpu.VMEM((CHUNK,), jnp.int32),
        pltpu.VMEM((CHUNK, EMB), jnp.float32),
        pltpu.SemaphoreType.DMA,
    ],
    compiler_params=pltpu.CompilerParams(use_tc_tiling_on_sc=False),
)
def _emb_lookup(idx_hbm, table_hbm, out_hbm, idx_v, rows_v, sem):
    wid = lax.axis_index("s") * 2 + lax.axis_index("c")
    base = wid * B_PER_W

    def chunk_body(ci, carry):
        off = base + ci * CHUNK
        pltpu.sync_copy(idx_hbm.at[pl.ds(off, CHUNK)], idx_v)
        pltpu.async_copy(table_hbm.at[idx_v], rows_v, sem).wait()

        def scale_body(r, c):
            rows_v[r, pl.ds(0, 16)] = rows_v[r, pl.ds(0, 16)] * SCALE
            rows_v[r, pl.ds(16, 16)] = rows_v[r, pl.ds(16, 16)] * SCALE
            return c

        lax.fori_loop(0, CHUNK, scale_body, 0)
        pltpu.sync_copy(rows_v, out_hbm.at[pl.ds(off, CHUNK)])
        return carry

    lax.fori_loop(0, N_CHUNKS, chunk_body, 0)


def kernel(tokens, table):
    idx = tokens.reshape(-1).astype(jnp.int32)
    out = _emb_lookup(idx, table)
    return out.reshape(tokens.shape + (EMB,))


# speedup vs baseline: 1.0814x; 1.0814x over previous
import functools
import math

import jax
import jax.numpy as jnp
from jax import lax
from jax.experimental import pallas as pl
from jax.experimental.pallas import tpu as pltpu
from jax.experimental.pallas import tpu_sc as plsc

EMB = 32
SCALE = math.sqrt(32.0)

B = 16384 * 50
NW = 32
B_PER_W = B // NW
CHUNK = 1280
N_CHUNKS = B_PER_W // CHUNK

_mesh = plsc.VectorSubcoreMesh(core_axis_name="c", subcore_axis_name="s")


@functools.partial(
    pl.kernel,
    mesh=_mesh,
    out_type=jax.ShapeDtypeStruct((B, EMB), jnp.float32),
    scratch_types=[
        pltpu.VMEM((2, CHUNK), jnp.int32),
        pltpu.VMEM((2, CHUNK, EMB), jnp.float32),
        pltpu.SemaphoreType.DMA((2,)),
    ],
    compiler_params=pltpu.CompilerParams(use_tc_tiling_on_sc=False),
)
def _emb_lookup(idx_hbm, table_hbm, out_hbm, idx_v, rows_v, gsem):
    wid = lax.axis_index("s") * 2 + lax.axis_index("c")
    base = wid * B_PER_W

    pltpu.sync_copy(idx_hbm.at[pl.ds(base, CHUNK)], idx_v.at[0])
    pltpu.async_copy(table_hbm.at[idx_v.at[0]], rows_v.at[0], gsem.at[0])

    def outer(g, carry):
        for b in (0, 1):
            ci = g * 2 + b
            nb = 1 - b

            @pl.when(ci + 1 < N_CHUNKS)
            def _():
                off_n = base + (ci + 1) * CHUNK
                pltpu.sync_copy(idx_hbm.at[pl.ds(off_n, CHUNK)], idx_v.at[nb])
                pltpu.async_copy(
                    table_hbm.at[idx_v.at[nb]], rows_v.at[nb], gsem.at[nb]
                )

            pltpu.make_async_copy(
                table_hbm.at[idx_v.at[b]], rows_v.at[b], gsem.at[b]
            ).wait()

            @plsc.parallel_loop(0, CHUNK, step=1, unroll=8)
            def _(r):
                rows_v[b, r, pl.ds(0, 16)] = rows_v[b, r, pl.ds(0, 16)] * SCALE
                rows_v[b, r, pl.ds(16, 16)] = rows_v[b, r, pl.ds(16, 16)] * SCALE

            off = base + ci * CHUNK
            pltpu.sync_copy(rows_v.at[b], out_hbm.at[pl.ds(off, CHUNK)])
        return carry

    lax.fori_loop(0, N_CHUNKS // 2, outer, 0)


def kernel(tokens, table):
    idx = tokens.reshape(-1).astype(jnp.int32)
    out = _emb_lookup(idx, table)
    return out.reshape(tokens.shape + (EMB,))


# speedup vs baseline: 1.6890x; 1.5619x over previous
import functools
import math

import jax
import jax.numpy as jnp
from jax import lax
from jax.experimental import pallas as pl
from jax.experimental.pallas import tpu as pltpu
from jax.experimental.pallas import tpu_sc as plsc

EMB = 32
SCALE = math.sqrt(32.0)

ROWS = 16384
COLS = 50
NW = 32
R_PER_W = ROWS // NW
TR = 16
N_CHUNKS = R_PER_W // TR
LOOK = TR * COLS

_mesh = plsc.VectorSubcoreMesh(core_axis_name="c", subcore_axis_name="s")


@functools.partial(
    pl.kernel,
    mesh=_mesh,
    out_type=jax.ShapeDtypeStruct((ROWS, COLS, EMB), jnp.float32),
    scratch_types=[
        pltpu.VMEM((2, TR, COLS), jnp.int32),
        pltpu.VMEM((2, TR, COLS, EMB), jnp.float32),
        pltpu.SemaphoreType.DMA((2,)),
    ],
    compiler_params=pltpu.CompilerParams(use_tc_tiling_on_sc=False),
)
def _emb_lookup(tok_hbm, table_hbm, out_hbm, idx_v, rows_v, gsem):
    wid = lax.axis_index("s") * 2 + lax.axis_index("c")
    base = wid * R_PER_W

    def fire(ci, b):
        r0 = base + ci * TR
        pltpu.sync_copy(tok_hbm.at[pl.ds(r0, TR)], idx_v.at[b])
        for r in range(TR):
            pltpu.async_copy(
                table_hbm.at[idx_v.at[b, r]], rows_v.at[b, r], gsem.at[b]
            )

    def drain(ci, b):
        r0 = base + ci * TR
        pltpu.make_async_copy(
            out_hbm.at[pl.ds(r0, TR)], rows_v.at[b], gsem.at[b]
        ).wait()

    fire(0, 0)

    def outer(g, carry):
        for b in (0, 1):
            ci = g * 2 + b
            nb = 1 - b

            @pl.when(ci + 1 < N_CHUNKS)
            def _():
                fire(ci + 1, nb)

            drain(ci, b)

            @plsc.parallel_loop(0, LOOK, step=1, unroll=8)
            def _(q):
                r = q // COLS
                c = q % COLS
                rows_v[b, r, c, pl.ds(0, 16)] = (
                    rows_v[b, r, c, pl.ds(0, 16)] * SCALE
                )
                rows_v[b, r, c, pl.ds(16, 16)] = (
                    rows_v[b, r, c, pl.ds(16, 16)] * SCALE
                )

            r0 = base + ci * TR
            pltpu.sync_copy(rows_v.at[b], out_hbm.at[pl.ds(r0, TR)])
        return carry

    lax.fori_loop(0, N_CHUNKS // 2, outer, 0)


def kernel(tokens, table):
    return _emb_lookup(tokens.astype(jnp.int32), table)


# speedup vs baseline: 1.7333x; 1.0263x over previous
import functools
import math

import jax
import jax.numpy as jnp
from jax import lax
from jax.experimental import pallas as pl
from jax.experimental.pallas import tpu as pltpu
from jax.experimental.pallas import tpu_sc as plsc

EMB = 32
SCALE = math.sqrt(32.0)

ROWS = 16384
COLS = 50
NW = 32
R_PER_W = ROWS // NW
TR = 16
N_CHUNKS = R_PER_W // TR
LOOK = TR * COLS
OUT_W = 128
PACK = OUT_W // EMB
OUT_ROWS = ROWS * COLS // PACK
OR_PER_CHUNK = LOOK // PACK

_mesh = plsc.VectorSubcoreMesh(core_axis_name="c", subcore_axis_name="s")


@functools.partial(
    pl.kernel,
    mesh=_mesh,
    out_type=jax.ShapeDtypeStruct((OUT_ROWS, OUT_W), jnp.float32),
    scratch_types=[
        pltpu.VMEM((2, TR, COLS), jnp.int32),
        pltpu.VMEM((2, LOOK, EMB), jnp.float32),
        pltpu.VMEM((2, OR_PER_CHUNK, OUT_W), jnp.float32),
        pltpu.SemaphoreType.DMA((2,)),
    ],
    compiler_params=pltpu.CompilerParams(use_tc_tiling_on_sc=False),
)
def _emb_lookup(tok_hbm, table_hbm, out_hbm, idx_v, rows_v, out_v, gsem):
    wid = lax.axis_index("s") * 2 + lax.axis_index("c")
    base = wid * R_PER_W

    def fire(ci, b):
        r0 = base + ci * TR
        pltpu.sync_copy(tok_hbm.at[pl.ds(r0, TR)], idx_v.at[b])
        for r in range(TR):
            pltpu.async_copy(
                table_hbm.at[idx_v.at[b, r]],
                rows_v.at[b, pl.ds(r * COLS, COLS)],
                gsem.at[b],
            )

    def drain(b):
        for r in range(TR):
            pltpu.make_async_copy(
                table_hbm.at[idx_v.at[b, r]],
                rows_v.at[b, pl.ds(r * COLS, COLS)],
                gsem.at[b],
            ).wait()

    fire(0, 0)

    def outer(g, carry):
        for b in (0, 1):
            ci = g * 2 + b
            nb = 1 - b

            @pl.when(ci + 1 < N_CHUNKS)
            def _():
                fire(ci + 1, nb)

            drain(b)

            @plsc.parallel_loop(0, LOOK, step=1, unroll=8)
            def _(q):
                fr = q // PACK
                c0 = (q % PACK) * EMB
                out_v[b, fr, pl.ds(c0, 16)] = rows_v[b, q, pl.ds(0, 16)] * SCALE
                out_v[b, fr, pl.ds(c0 + 16, 16)] = (
                    rows_v[b, q, pl.ds(16, 16)] * SCALE
                )

            f0 = wid * (R_PER_W * COLS // PACK) + ci * OR_PER_CHUNK
            pltpu.sync_copy(out_v.at[b], out_hbm.at[pl.ds(f0, OR_PER_CHUNK)])
        return carry

    lax.fori_loop(0, N_CHUNKS // 2, outer, 0)


def kernel(tokens, table):
    out = _emb_lookup(tokens.astype(jnp.int32), table)
    return out.reshape(ROWS, COLS, EMB)
